# Initial kernel scaffold; baseline (speedup 1.0000x reference)
import jax
import jax.numpy as jnp
from jax.experimental import pallas as pl


def kernel(x, W_enc, b_enc, W_dec, b_dec, codebooks):
    raise NotImplementedError("write your pallas kernel here")


# speedup vs baseline: 1.0401x; 1.0401x over previous
import functools
import math

import jax
import jax.numpy as jnp
from jax import lax
from jax.experimental import pallas as pl
from jax.experimental.pallas import tpu as pltpu

_P = 4
_L = 4
_K = 2048
_D = 48
_LMBDA = 0.01
_BN = 512


def _patchify(x, p):
    B, C, H, W = x.shape
    x = x.reshape(B, C, H // p, p, W // p, p)
    return jnp.transpose(x, (0, 2, 4, 1, 3, 5)).reshape(
        B * (H // p) * (W // p), C * p * p)


def _unpatchify(v, shape, p):
    B, C, H, W = shape
    v = v.reshape(B, H // p, W // p, C, p, p)
    return jnp.transpose(v, (0, 3, 1, 4, 2, 5)).reshape(B, C, H, W)


def _vq_body(v_ref, we_ref, be_ref, wd_ref, bd_ref, cb_ref,
             y_ref, lik_ref, vq_ref, rate_ref, mse_ref, c2_ref):
    i = pl.program_id(0)

    @pl.when(i == 0)
    def _init():
        cb = cb_ref[...]
        c2_ref[...] = jnp.sum(cb * cb, axis=-1)

    v = v_ref[...]
    z = jnp.tanh(
        lax.dot_general(v, we_ref[...], (((1,), (0,)), ((), ())),
                        preferred_element_type=jnp.float32) + be_ref[...])
    r = z
    qs = jnp.zeros_like(z)
    rate = jnp.zeros((_BN,), jnp.float32)
    vq = jnp.float32(0.0)
    for l in range(_L):
        C = cb_ref[l]
        prod = lax.dot_general(r, C, (((1,), (1,)), ((), ())),
                               preferred_element_type=jnp.float32)
        e = c2_ref[l][None, :] - 2.0 * prod
        m = jnp.min(e, axis=1)
        idx = jnp.argmin(e, axis=1)
        s = jnp.sum(jnp.exp(m[:, None] - e), axis=1)
        rate = rate + jnp.log2(s)
        r2 = jnp.sum(r * r, axis=1)
        vq = vq + jnp.sum(r2 + m)
        oh = (lax.broadcasted_iota(jnp.int32, (_BN, _K), 1)
              == idx[:, None]).astype(jnp.float32)
        q = lax.dot_general(oh, C, (((1,), (0,)), ((), ())),
                            preferred_element_type=jnp.float32,
                            precision=lax.Precision.HIGHEST)
        qs = qs + q
        r = r - q
    y = lax.dot_general(qs, wd_ref[...], (((1,), (0,)), ((), ())),
                        preferred_element_type=jnp.float32) + bd_ref[...]
    y_ref[...] = y
    lik_ref[...] = jnp.exp2(-rate)
    dv = y - v
    blk_vq = vq
    blk_rate = jnp.sum(rate)
    blk_mse = jnp.sum(dv * dv)

    @pl.when(i == 0)
    def _store():
        vq_ref[...] = blk_vq[None, None]
        rate_ref[...] = blk_rate[None, None]
        mse_ref[...] = blk_mse[None, None]

    @pl.when(i > 0)
    def _acc():
        vq_ref[...] += blk_vq[None, None]
        rate_ref[...] += blk_rate[None, None]
        mse_ref[...] += blk_mse[None, None]


@functools.partial(jax.jit, static_argnames=())
def kernel(x, W_enc, b_enc, W_dec, b_dec, codebooks):
    shape = x.shape
    v = _patchify(x, _P)
    n = v.shape[0]
    nblk = n // _BN

    y, lik, vqs, rates, mses = pl.pallas_call(
        _vq_body,
        grid=(nblk,),
        in_specs=[
            pl.BlockSpec((_BN, _D), lambda i: (i, 0)),
            pl.BlockSpec((_D, _D), lambda i: (0, 0)),
            pl.BlockSpec((1, _D), lambda i: (0, 0)),
            pl.BlockSpec((_D, _D), lambda i: (0, 0)),
            pl.BlockSpec((1, _D), lambda i: (0, 0)),
            pl.BlockSpec((_L, _K, _D), lambda i: (0, 0, 0)),
        ],
        out_specs=[
            pl.BlockSpec((_BN, _D), lambda i: (i, 0)),
            pl.BlockSpec((_BN,), lambda i: (i,)),
            pl.BlockSpec((1, 1), lambda i: (0, 0)),
            pl.BlockSpec((1, 1), lambda i: (0, 0)),
            pl.BlockSpec((1, 1), lambda i: (0, 0)),
        ],
        out_shape=[
            jax.ShapeDtypeStruct((n, _D), jnp.float32),
            jax.ShapeDtypeStruct((n,), jnp.float32),
            jax.ShapeDtypeStruct((1, 1), jnp.float32),
            jax.ShapeDtypeStruct((1, 1), jnp.float32),
            jax.ShapeDtypeStruct((1, 1), jnp.float32),
        ],
        scratch_shapes=[pltpu.VMEM((_L, _K), jnp.float32)],
    )(v, W_enc, b_enc.reshape(1, _D), W_dec, b_dec.reshape(1, _D), codebooks)

    x_hat = _unpatchify(y, shape, _P)
    rate = rates[0, 0] / n
    mse = mses[0, 0] / (n * _D)
    vq_loss = 1.25 * vqs[0, 0] / (n * _D)
    rd_loss = rate + _LMBDA * mse * (255.0 ** 2)
    loss = rd_loss + vq_loss
    return (x_hat, lik, loss, rd_loss, vq_loss)


# speedup vs baseline: 2.1862x; 2.1020x over previous
import functools

import jax
import jax.numpy as jnp
from jax import lax
from jax.experimental import pallas as pl
from jax.experimental.pallas import tpu as pltpu

_P = 4
_L = 4
_K = 2048
_D = 48
_LMBDA = 0.01
_BN = 1024
_G = 256
_S = _K // _G


def _patchify(x, p):
    B, C, H, W = x.shape
    x = x.reshape(B, C, H // p, p, W // p, p)
    return jnp.transpose(x, (0, 2, 4, 1, 3, 5)).reshape(
        B * (H // p) * (W // p), C * p * p)


def _unpatchify(v, shape, p):
    B, C, H, W = shape
    v = v.reshape(B, H // p, W // p, C, p, p)
    return jnp.transpose(v, (0, 3, 1, 4, 2, 5)).reshape(B, C, H, W)


def _vq_body(vt_ref, we_ref, bet_ref, wd_ref, bdt_ref, cb_ref,
             yt_ref, lik_ref, vq_ref, rate_ref, mse_ref,
             ca_ref, ch_ref, cm_ref, cl_ref):
    i = pl.program_id(0)

    @pl.when(i == 0)
    def _init():
        cb = cb_ref[...]
        c2 = jnp.sum(cb * cb, axis=-1)
        c2h = c2.astype(jnp.bfloat16).astype(jnp.float32)
        c2r = c2 - c2h
        c2m = c2r.astype(jnp.bfloat16).astype(jnp.float32)
        c2l = c2r - c2m
        ca_ref[...] = jnp.concatenate(
            [-2.0 * cb, c2h[..., None], c2m[..., None], c2l[..., None]],
            axis=-1)
        hi = cb.astype(jnp.bfloat16)
        rem = cb - hi.astype(jnp.float32)
        mid = rem.astype(jnp.bfloat16)
        lo = (rem - mid.astype(jnp.float32)).astype(jnp.bfloat16)

        def _pack(p):
            return jnp.concatenate(
                [p[:, sl * _G:(sl + 1) * _G, :] for sl in range(_S)], axis=-1)
        ch_ref[...] = _pack(hi)
        cm_ref[...] = _pack(mid)
        cl_ref[...] = _pack(lo)

    vt = vt_ref[...]
    z = jnp.tanh(
        lax.dot_general(we_ref[...], vt, (((0,), (0,)), ((), ())),
                        preferred_element_type=jnp.float32) + bet_ref[...])
    r = z
    qs = jnp.zeros_like(z)
    rate = jnp.zeros((_BN,), jnp.float32)
    vq = jnp.float32(0.0)
    ones3 = jnp.ones((3, _BN), jnp.float32)
    for l in range(_L):
        r_aug = jnp.concatenate([r, ones3], axis=0)
        e = lax.dot_general(ca_ref[l], r_aug, (((1,), (0,)), ((), ())),
                            preferred_element_type=jnp.float32)
        m = jnp.min(e, axis=0)
        idx = jnp.argmin(e, axis=0)
        s = jnp.sum(jnp.exp(m[None, :] - e), axis=0)
        rate = rate + jnp.log2(s)
        r2 = jnp.sum(r * r, axis=0)
        vq = vq + jnp.sum(r2) + jnp.sum(m)
        grp = idx & (_G - 1)
        slot = idx >> 8
        oht = (lax.broadcasted_iota(jnp.int32, (_G, _BN), 0)
               == grp[None, :]).astype(jnp.bfloat16)
        dn = (((0,), (0,)), ((), ()))
        t = (lax.dot_general(cl_ref[l], oht, dn,
                             preferred_element_type=jnp.float32)
             + lax.dot_general(cm_ref[l], oht, dn,
                               preferred_element_type=jnp.float32)
             + lax.dot_general(ch_ref[l], oht, dn,
                               preferred_element_type=jnp.float32))
        q = jnp.zeros((_D, _BN), jnp.float32)
        for sslot in range(_S):
            q = q + jnp.where((slot == sslot)[None, :],
                              t[sslot * _D:(sslot + 1) * _D, :], 0.0)
        qs = qs + q
        r = r - q
    yt = lax.dot_general(wd_ref[...], qs, (((0,), (0,)), ((), ())),
                         preferred_element_type=jnp.float32) + bdt_ref[...]
    yt_ref[...] = yt
    lik_ref[...] = jnp.exp2(-rate)
    dv = yt - vt
    blk_vq = vq
    blk_rate = jnp.sum(rate)
    blk_mse = jnp.sum(dv * dv)

    @pl.when(i == 0)
    def _store():
        vq_ref[...] = blk_vq[None, None]
        rate_ref[...] = blk_rate[None, None]
        mse_ref[...] = blk_mse[None, None]

    @pl.when(i > 0)
    def _acc():
        vq_ref[...] += blk_vq[None, None]
        rate_ref[...] += blk_rate[None, None]
        mse_ref[...] += blk_mse[None, None]


@functools.partial(jax.jit, static_argnames=())
def kernel(x, W_enc, b_enc, W_dec, b_dec, codebooks):
    shape = x.shape
    v = _patchify(x, _P)
    n = v.shape[0]
    nblk = n // _BN
    vt = v.T

    yt, lik, vqs, rates, mses = pl.pallas_call(
        _vq_body,
        grid=(nblk,),
        in_specs=[
            pl.BlockSpec((_D, _BN), lambda i: (0, i)),
            pl.BlockSpec((_D, _D), lambda i: (0, 0)),
            pl.BlockSpec((_D, 1), lambda i: (0, 0)),
            pl.BlockSpec((_D, _D), lambda i: (0, 0)),
            pl.BlockSpec((_D, 1), lambda i: (0, 0)),
            pl.BlockSpec((_L, _K, _D), lambda i: (0, 0, 0)),
        ],
        out_specs=[
            pl.BlockSpec((_D, _BN), lambda i: (0, i)),
            pl.BlockSpec((_BN,), lambda i: (i,)),
            pl.BlockSpec((1, 1), lambda i: (0, 0)),
            pl.BlockSpec((1, 1), lambda i: (0, 0)),
            pl.BlockSpec((1, 1), lambda i: (0, 0)),
        ],
        out_shape=[
            jax.ShapeDtypeStruct((_D, n), jnp.float32),
            jax.ShapeDtypeStruct((n,), jnp.float32),
            jax.ShapeDtypeStruct((1, 1), jnp.float32),
            jax.ShapeDtypeStruct((1, 1), jnp.float32),
            jax.ShapeDtypeStruct((1, 1), jnp.float32),
        ],
        scratch_shapes=[
            pltpu.VMEM((_L, _K, _D + 3), jnp.float32),
            pltpu.VMEM((_L, _G, _S * _D), jnp.bfloat16),
            pltpu.VMEM((_L, _G, _S * _D), jnp.bfloat16),
            pltpu.VMEM((_L, _G, _S * _D), jnp.bfloat16),
        ],
    )(vt, W_enc, b_enc.reshape(_D, 1), W_dec, b_dec.reshape(_D, 1),
      codebooks)

    x_hat = _unpatchify(yt.T, shape, _P)
    rate = rates[0, 0] / n
    mse = mses[0, 0] / (n * _D)
    vq_loss = 1.25 * vqs[0, 0] / (n * _D)
    rd_loss = rate + _LMBDA * mse * (255.0 ** 2)
    loss = rd_loss + vq_loss
    return (x_hat, lik, loss, rd_loss, vq_loss)
